# baseline (speedup 1.0000x reference)
import functools

import jax
import jax.numpy as jnp
from jax import lax
from jax.experimental import pallas as pl
from jax.experimental.pallas import tpu as pltpu
from jax.experimental.pallas import tpu_sc as plsc

BB, SS, DD = 2, 2048, 768
EE = 8
FF = 1024
NTOK = BB * SS
NW = 32
CHUNK = NTOK // NW
TROW = 256
NTILES = 40
XROWS = NTILES * TROW
EOTPAD = 48


def _router_body(x_ref, wr_ref, e0_ref, e1_ref, g0_ref, g1_ref, cnt_ref):
    x = x_ref[...]
    logits = jnp.dot(x, wr_ref[...], preferred_element_type=jnp.float32)
    probs = jax.nn.softmax(logits, axis=-1)
    i1 = jnp.argmax(probs, axis=-1)[:, None]
    p1 = jnp.max(probs, axis=-1, keepdims=True)
    cols = lax.broadcasted_iota(jnp.int32, probs.shape, 1)
    masked = jnp.where(cols == i1, -jnp.inf, probs)
    i2 = jnp.argmax(masked, axis=-1)[:, None]
    p2 = jnp.max(masked, axis=-1, keepdims=True)
    e2 = jnp.exp(p2 - p1)
    e0_ref[...] = i1
    e1_ref[...] = i2
    g0_ref[...] = 1.0 / (1.0 + e2)
    g1_ref[...] = e2 / (1.0 + e2)
    onehot = ((cols == i1) | (cols == i2)).astype(jnp.float32)
    chunk_of = lax.broadcasted_iota(jnp.int32, (NTOK, NW), 0) // CHUNK
    wcol = lax.broadcasted_iota(jnp.int32, (NTOK, NW), 1)
    ind = (chunk_of == wcol).astype(jnp.float32)
    cnt = lax.dot_general(ind, onehot, (((0,), (0,)), ((), ())),
                          preferred_element_type=jnp.float32)
    cnt_ref[...] = cnt.astype(jnp.int32)


def _router(x, Wr):
    return pl.pallas_call(
        _router_body,
        grid=(1,),
        in_specs=[
            pl.BlockSpec((NTOK, DD), lambda i: (0, 0)),
            pl.BlockSpec((DD, EE), lambda i: (0, 0)),
        ],
        out_specs=[
            pl.BlockSpec((NTOK, 1), lambda i: (0, 0)),
            pl.BlockSpec((NTOK, 1), lambda i: (0, 0)),
            pl.BlockSpec((NTOK, 1), lambda i: (0, 0)),
            pl.BlockSpec((NTOK, 1), lambda i: (0, 0)),
            pl.BlockSpec((NW, EE), lambda i: (0, 0)),
        ],
        out_shape=[
            jax.ShapeDtypeStruct((NTOK, 1), jnp.int32),
            jax.ShapeDtypeStruct((NTOK, 1), jnp.int32),
            jax.ShapeDtypeStruct((NTOK, 1), jnp.float32),
            jax.ShapeDtypeStruct((NTOK, 1), jnp.float32),
            jax.ShapeDtypeStruct((NW, EE), jnp.int32),
        ],
        compiler_params=pltpu.CompilerParams(
            dimension_semantics=("arbitrary",),
        ),
    )(x, Wr)


def _route_body(e0_hbm, e1_hbm, cnt_hbm, x_hbm, g0_hbm, g1_hbm,
                xs_hbm, inv0_hbm, inv1_hbm, eot_hbm, gsort_hbm,
                e0_v, e1_v, cnt_v, pos0_v, pos1_v, xbuf, eot_v,
                g0_v, g1_v, grows, sem):
    info = plsc.get_sparse_core_info()
    nc = info.num_cores
    w = lax.axis_index("s") * nc + lax.axis_index("c")
    t0 = w * CHUNK
    pltpu.sync_copy(e0_hbm.at[pl.ds(t0, CHUNK)], e0_v)
    pltpu.sync_copy(e1_hbm.at[pl.ds(t0, CHUNK)], e1_v)
    pltpu.sync_copy(g0_hbm.at[pl.ds(t0, CHUNK)], g0_v)
    pltpu.sync_copy(g1_hbm.at[pl.ds(t0, CHUNK)], g1_v)
    pltpu.sync_copy(cnt_hbm, cnt_v)

    lanes = lax.iota(jnp.int32, 16)
    starts = []
    nt = []
    base = jnp.int32(0)
    for e in range(EE):
        col_a = plsc.load_gather(cnt_v, [lanes * EE + e])
        col_b = plsc.load_gather(cnt_v, [(lanes + 16) * EE + e])
        c_e = jnp.sum(col_a) + jnp.sum(col_b)
        pref = (jnp.sum(jnp.where(lanes < w, col_a, 0))
                + jnp.sum(jnp.where(lanes + 16 < w, col_b, 0)))
        starts.append(base + pref)
        nt_e = (c_e + (TROW - 1)) // TROW
        nt.append(nt_e)
        base = base + nt_e * TROW

    @pl.when(w == 0)
    def _():
        vals = [jnp.full((16,), EE - 1, jnp.int32) for _ in range(EOTPAD // 16)]
        toff = jnp.int32(0)
        for e in range(EE):
            for k in range(EOTPAD // 16):
                tv = lanes + 16 * k
                m = (tv >= toff) & (tv < toff + nt[e])
                vals[k] = jnp.where(m, e, vals[k])
            toff = toff + nt[e]
        for k in range(EOTPAD // 16):
            eot_v[pl.ds(16 * k, 16)] = vals[k]
        pltpu.sync_copy(eot_v, eot_hbm)

    for ev_ref, pos_ref in ((e0_v, pos0_v), (e1_v, pos1_v)):
        for v in range(CHUNK // 16):
            ev = ev_ref[pl.ds(v * 16, 16)]
            pos = jnp.zeros((16,), jnp.int32)
            for e in range(EE):
                m = ev == e
                pc = plsc.cumsum(jnp.where(m, 1, 0).astype(jnp.int32))
                pos = jnp.where(m, starts[e] + pc - 1, pos)
                starts[e] = starts[e] + jnp.max(pc)
            pos_ref[v // 4, pl.ds((v % 4) * 16, 16)] = pos

    pltpu.sync_copy(pos0_v, inv0_hbm.at[w])
    pltpu.sync_copy(pos1_v, inv1_hbm.at[w])

    for gch, pos_v in ((g0_v, pos0_v), (g1_v, pos1_v)):
        for c in range(2):
            def gbody(j, carry, gch=gch, c=c):
                gv = plsc.load_gather(
                    gch, [jnp.full((16,), c * 64 + j, jnp.int32)])
                for k in range(8):
                    grows[j, pl.ds(k * 16, 16)] = gv
                return carry
            lax.fori_loop(0, 64, gbody, jnp.int32(0))
            pltpu.async_copy(grows, gsort_hbm.at[pos_v.at[c]], sem).wait()

    for c in range(2):
        pltpu.sync_copy(x_hbm.at[pl.ds(t0 + c * 64, 64)], xbuf)
        pltpu.async_copy(xbuf, xs_hbm.at[pos0_v.at[c]], sem).wait()
        pltpu.async_copy(xbuf, xs_hbm.at[pos1_v.at[c]], sem).wait()


def _route(e0, e1, cnt, x, g0, g1):
    mesh = plsc.VectorSubcoreMesh(core_axis_name="c", subcore_axis_name="s")
    fn = pl.kernel(
        _route_body,
        mesh=mesh,
        out_type=[
            jax.ShapeDtypeStruct((XROWS, DD), jnp.float32),
            jax.ShapeDtypeStruct((NW, 2, 64), jnp.int32),
            jax.ShapeDtypeStruct((NW, 2, 64), jnp.int32),
            jax.ShapeDtypeStruct((EOTPAD,), jnp.int32),
            jax.ShapeDtypeStruct((XROWS, 128), jnp.float32),
        ],
        scratch_types=[
            pltpu.VMEM((CHUNK,), jnp.int32),
            pltpu.VMEM((CHUNK,), jnp.int32),
            pltpu.VMEM((NW * EE,), jnp.int32),
            pltpu.VMEM((2, 64), jnp.int32),
            pltpu.VMEM((2, 64), jnp.int32),
            pltpu.VMEM((64, DD), jnp.float32),
            pltpu.VMEM((EOTPAD,), jnp.int32),
            pltpu.VMEM((CHUNK,), jnp.float32),
            pltpu.VMEM((CHUNK,), jnp.float32),
            pltpu.VMEM((64, 128), jnp.float32),
            pltpu.SemaphoreType.DMA,
        ],
        compiler_params=pltpu.CompilerParams(needs_layout_passes=False),
    )
    return fn(e0, e1, cnt, x, g0, g1)


CSTEPS = 10
TPS = NTILES // CSTEPS
CROWS = TPS * TROW


def _ffn_body(eot_ref, xs_ref, w1_ref, b1_ref, w2_ref, b2_ref, gs_ref, ys_ref):
    i = pl.program_id(0)
    for t in range(TPS):
        e = eot_ref[i * TPS + t]
        x = xs_ref[pl.ds(t * TROW, TROW), :]
        h = jnp.dot(x, w1_ref[e], preferred_element_type=jnp.float32) + b1_ref[e]
        h = h * 0.5 * (1.0 + lax.erf(h * 0.7071067811865476))
        y = jnp.dot(h, w2_ref[e], preferred_element_type=jnp.float32) + b2_ref[e]
        gs = gs_ref[pl.ds(t * TROW, TROW), 0:1]
        ys_ref[pl.ds(t * TROW, TROW), :] = y * gs


def _ffn(eot, xs, W1, b1, W2, b2, gsort):
    grid_spec = pltpu.PrefetchScalarGridSpec(
        num_scalar_prefetch=1,
        grid=(CSTEPS,),
        in_specs=[
            pl.BlockSpec((CROWS, DD), lambda i, eot: (i, 0)),
            pl.BlockSpec((EE, DD, FF), lambda i, eot: (0, 0, 0)),
            pl.BlockSpec((EE, FF), lambda i, eot: (0, 0)),
            pl.BlockSpec((EE, FF, DD), lambda i, eot: (0, 0, 0)),
            pl.BlockSpec((EE, DD), lambda i, eot: (0, 0)),
            pl.BlockSpec((CROWS, 128), lambda i, eot: (i, 0)),
        ],
        out_specs=pl.BlockSpec((CROWS, DD), lambda i, eot: (i, 0)),
    )
    return pl.pallas_call(
        _ffn_body,
        grid_spec=grid_spec,
        out_shape=jax.ShapeDtypeStruct((XROWS, DD), jnp.float32),
        compiler_params=pltpu.CompilerParams(
            dimension_semantics=("arbitrary",),
            vmem_limit_bytes=134217728,
        ),
    )(eot, xs, W1, b1, W2, b2, gsort)


def _combine_body(ys_hbm, inv0_hbm, inv1_hbm, out_hbm,
                  i0_v, i1_v, buf, buf2, sem):
    info = plsc.get_sparse_core_info()
    nc = info.num_cores
    w = lax.axis_index("s") * nc + lax.axis_index("c")
    t0 = w * CHUNK
    pltpu.sync_copy(inv0_hbm.at[w], i0_v)
    pltpu.sync_copy(inv1_hbm.at[w], i1_v)
    for c in range(2):
        pltpu.async_copy(ys_hbm.at[i0_v.at[c]], buf, sem).wait()
        pltpu.async_copy(ys_hbm.at[i1_v.at[c]], buf2, sem).wait()

        def body(r, carry):
            for k in range(DD // 16):
                buf[r, pl.ds(k * 16, 16)] = (buf[r, pl.ds(k * 16, 16)]
                                             + buf2[r, pl.ds(k * 16, 16)])
            return carry

        lax.fori_loop(0, 64, body, jnp.int32(0))
        pltpu.sync_copy(buf, out_hbm.at[pl.ds(t0 + c * 64, 64)])


def _combine(ys, inv0, inv1):
    mesh = plsc.VectorSubcoreMesh(core_axis_name="c", subcore_axis_name="s")
    fn = pl.kernel(
        _combine_body,
        mesh=mesh,
        out_type=jax.ShapeDtypeStruct((NTOK, DD), jnp.float32),
        scratch_types=[
            pltpu.VMEM((2, 64), jnp.int32),
            pltpu.VMEM((2, 64), jnp.int32),
            pltpu.VMEM((64, DD), jnp.float32),
            pltpu.VMEM((64, DD), jnp.float32),
            pltpu.SemaphoreType.DMA,
        ],
        compiler_params=pltpu.CompilerParams(needs_layout_passes=False),
    )
    return fn(ys, inv0, inv1)


def kernel(inputs, Wr, W1, b1, W2, b2):
    x = inputs.reshape(NTOK, DD)
    e0, e1, g0, g1, cnt = _router(x, Wr)
    xs, inv0, inv1, eot, gsort = _route(
        e0.reshape(NTOK), e1.reshape(NTOK), cnt.reshape(NW * EE), x,
        g0.reshape(NTOK), g1.reshape(NTOK))
    ys = _ffn(eot, xs, W1, b1, W2, b2, gsort)
    out = _combine(ys, inv0, inv1)
    return out.reshape(BB, SS, DD)


# speedup vs baseline: 1.0074x; 1.0074x over previous
import functools

import jax
import jax.numpy as jnp
from jax import lax
from jax.experimental import pallas as pl
from jax.experimental.pallas import tpu as pltpu
from jax.experimental.pallas import tpu_sc as plsc

BB, SS, DD = 2, 2048, 768
EE = 8
FF = 1024
NTOK = BB * SS
NW = 32
CHUNK = NTOK // NW
TROW = 256
NTILES = 40
XROWS = NTILES * TROW
EOTPAD = 48


def _router_body(x_ref, wr_ref, e0_ref, e1_ref, g0_ref, g1_ref, cnt_ref):
    x = x_ref[...]
    logits = jnp.dot(x, wr_ref[...], preferred_element_type=jnp.float32)
    probs = jax.nn.softmax(logits, axis=-1)
    i1 = jnp.argmax(probs, axis=-1)[:, None]
    p1 = jnp.max(probs, axis=-1, keepdims=True)
    cols = lax.broadcasted_iota(jnp.int32, probs.shape, 1)
    masked = jnp.where(cols == i1, -jnp.inf, probs)
    i2 = jnp.argmax(masked, axis=-1)[:, None]
    p2 = jnp.max(masked, axis=-1, keepdims=True)
    e2 = jnp.exp(p2 - p1)
    e0_ref[...] = i1
    e1_ref[...] = i2
    g0_ref[...] = 1.0 / (1.0 + e2)
    g1_ref[...] = e2 / (1.0 + e2)
    onehot = ((cols == i1) | (cols == i2)).astype(jnp.float32)
    chunk_of = lax.broadcasted_iota(jnp.int32, (NTOK, NW), 0) // CHUNK
    wcol = lax.broadcasted_iota(jnp.int32, (NTOK, NW), 1)
    ind = (chunk_of == wcol).astype(jnp.float32)
    cnt = lax.dot_general(ind, onehot, (((0,), (0,)), ((), ())),
                          preferred_element_type=jnp.float32)
    cnt_ref[...] = cnt.astype(jnp.int32)


def _router(x, Wr):
    return pl.pallas_call(
        _router_body,
        grid=(1,),
        in_specs=[
            pl.BlockSpec((NTOK, DD), lambda i: (0, 0)),
            pl.BlockSpec((DD, EE), lambda i: (0, 0)),
        ],
        out_specs=[
            pl.BlockSpec((NTOK, 1), lambda i: (0, 0)),
            pl.BlockSpec((NTOK, 1), lambda i: (0, 0)),
            pl.BlockSpec((NTOK, 1), lambda i: (0, 0)),
            pl.BlockSpec((NTOK, 1), lambda i: (0, 0)),
            pl.BlockSpec((NW, EE), lambda i: (0, 0)),
        ],
        out_shape=[
            jax.ShapeDtypeStruct((NTOK, 1), jnp.int32),
            jax.ShapeDtypeStruct((NTOK, 1), jnp.int32),
            jax.ShapeDtypeStruct((NTOK, 1), jnp.float32),
            jax.ShapeDtypeStruct((NTOK, 1), jnp.float32),
            jax.ShapeDtypeStruct((NW, EE), jnp.int32),
        ],
        compiler_params=pltpu.CompilerParams(
            dimension_semantics=("arbitrary",),
            fuse_transposed_lhs_in_matmul=True,
        ),
    )(x, Wr)


def _route_body(e0_hbm, e1_hbm, cnt_hbm, x_hbm,
                xs_hbm, inv0_hbm, inv1_hbm, eot_hbm,
                e0_v, e1_v, cnt_v, pos0_v, pos1_v, xbuf, eot_v, sem):
    info = plsc.get_sparse_core_info()
    nc = info.num_cores
    w = lax.axis_index("s") * nc + lax.axis_index("c")
    t0 = w * CHUNK
    pltpu.sync_copy(e0_hbm.at[pl.ds(t0, CHUNK)], e0_v)
    pltpu.sync_copy(e1_hbm.at[pl.ds(t0, CHUNK)], e1_v)
    pltpu.sync_copy(cnt_hbm, cnt_v)

    lanes = lax.iota(jnp.int32, 16)
    starts = []
    nt = []
    base = jnp.int32(0)
    for e in range(EE):
        col_a = plsc.load_gather(cnt_v, [lanes * EE + e])
        col_b = plsc.load_gather(cnt_v, [(lanes + 16) * EE + e])
        c_e = jnp.sum(col_a) + jnp.sum(col_b)
        pref = (jnp.sum(jnp.where(lanes < w, col_a, 0))
                + jnp.sum(jnp.where(lanes + 16 < w, col_b, 0)))
        starts.append(base + pref)
        nt_e = (c_e + (TROW - 1)) // TROW
        nt.append(nt_e)
        base = base + nt_e * TROW

    @pl.when(w == 0)
    def _():
        vals = [jnp.full((16,), EE - 1, jnp.int32) for _ in range(EOTPAD // 16)]
        toff = jnp.int32(0)
        for e in range(EE):
            for k in range(EOTPAD // 16):
                tv = lanes + 16 * k
                m = (tv >= toff) & (tv < toff + nt[e])
                vals[k] = jnp.where(m, e, vals[k])
            toff = toff + nt[e]
        kx, lx = divmod(NTILES, 16)
        vals[kx] = jnp.where(lanes == lx, toff, vals[kx])
        for k in range(EOTPAD // 16):
            eot_v[pl.ds(16 * k, 16)] = vals[k]
        pltpu.sync_copy(eot_v, eot_hbm)

    for ev_ref, pos_ref in ((e0_v, pos0_v), (e1_v, pos1_v)):
        for v in range(CHUNK // 16):
            ev = ev_ref[pl.ds(v * 16, 16)]
            pos = jnp.zeros((16,), jnp.int32)
            for e in range(EE):
                m = ev == e
                pc = plsc.cumsum(jnp.where(m, 1, 0).astype(jnp.int32))
                pos = jnp.where(m, starts[e] + pc - 1, pos)
                starts[e] = starts[e] + jnp.max(pc)
            pos_ref[v // 4, pl.ds((v % 4) * 16, 16)] = pos

    pltpu.sync_copy(pos0_v, inv0_hbm.at[w])
    pltpu.sync_copy(pos1_v, inv1_hbm.at[w])

    for c in range(2):
        pltpu.sync_copy(x_hbm.at[pl.ds(t0 + c * 64, 64)], xbuf)
        pltpu.async_copy(xbuf, xs_hbm.at[pos0_v.at[c]], sem).wait()
        pltpu.async_copy(xbuf, xs_hbm.at[pos1_v.at[c]], sem).wait()


def _route(e0, e1, cnt, x):
    mesh = plsc.VectorSubcoreMesh(core_axis_name="c", subcore_axis_name="s")
    fn = pl.kernel(
        _route_body,
        mesh=mesh,
        out_type=[
            jax.ShapeDtypeStruct((XROWS, DD), jnp.float32),
            jax.ShapeDtypeStruct((NW, 2, 64), jnp.int32),
            jax.ShapeDtypeStruct((NW, 2, 64), jnp.int32),
            jax.ShapeDtypeStruct((EOTPAD,), jnp.int32),
        ],
        scratch_types=[
            pltpu.VMEM((CHUNK,), jnp.int32),
            pltpu.VMEM((CHUNK,), jnp.int32),
            pltpu.VMEM((NW * EE,), jnp.int32),
            pltpu.VMEM((2, 64), jnp.int32),
            pltpu.VMEM((2, 64), jnp.int32),
            pltpu.VMEM((64, DD), jnp.float32),
            pltpu.VMEM((EOTPAD,), jnp.int32),
            pltpu.SemaphoreType.DMA,
        ],
        compiler_params=pltpu.CompilerParams(needs_layout_passes=False),
    )
    return fn(e0, e1, cnt, x)


CSTEPS = 10
TPS = NTILES // CSTEPS
CROWS = TPS * TROW


def _ffn_body(eot_ref, xs_ref, w1_ref, b1_ref, w2_ref, b2_ref, ys_ref):
    i = pl.program_id(0)
    used = eot_ref[NTILES]
    for t in range(TPS):
        @pl.when(i * TPS + t < used)
        def _(t=t):
            e = eot_ref[i * TPS + t]
            x = xs_ref[pl.ds(t * TROW, TROW), :]
            h = (jnp.dot(x, w1_ref[e], preferred_element_type=jnp.float32)
                 + b1_ref[e])
            h = h * 0.5 * (1.0 + lax.erf(h * 0.7071067811865476))
            ys_ref[pl.ds(t * TROW, TROW), :] = (
                jnp.dot(h, w2_ref[e], preferred_element_type=jnp.float32)
                + b2_ref[e])


def _ffn(eot, xs, W1, b1, W2, b2):
    grid_spec = pltpu.PrefetchScalarGridSpec(
        num_scalar_prefetch=1,
        grid=(CSTEPS,),
        in_specs=[
            pl.BlockSpec((CROWS, DD), lambda i, eot: (i, 0)),
            pl.BlockSpec((EE, DD, FF), lambda i, eot: (0, 0, 0)),
            pl.BlockSpec((EE, FF), lambda i, eot: (0, 0)),
            pl.BlockSpec((EE, FF, DD), lambda i, eot: (0, 0, 0)),
            pl.BlockSpec((EE, DD), lambda i, eot: (0, 0)),
        ],
        out_specs=pl.BlockSpec((CROWS, DD), lambda i, eot: (i, 0)),
    )
    return pl.pallas_call(
        _ffn_body,
        grid_spec=grid_spec,
        out_shape=jax.ShapeDtypeStruct((XROWS, DD), jnp.float32),
        compiler_params=pltpu.CompilerParams(
            dimension_semantics=("arbitrary",),
            vmem_limit_bytes=134217728,
        ),
    )(eot, xs, W1, b1, W2, b2)


def _combine_body(ys_hbm, inv0_hbm, inv1_hbm, g0_hbm, g1_hbm, out_hbm,
                  i0_v, i1_v, g0_v, g1_v, buf_e, buf_o, sem):
    info = plsc.get_sparse_core_info()
    nc = info.num_cores
    w = lax.axis_index("s") * nc + lax.axis_index("c")
    t0 = w * CHUNK
    pltpu.sync_copy(inv0_hbm.at[w], i0_v)
    pltpu.sync_copy(inv1_hbm.at[w], i1_v)
    pltpu.sync_copy(g0_hbm.at[pl.ds(t0, CHUNK)], g0_v)
    pltpu.sync_copy(g1_hbm.at[pl.ds(t0, CHUNK)], g1_v)
    for c in range(2):
        pltpu.async_copy(ys_hbm.at[i0_v.at[c]], buf_e, sem).wait()
        pltpu.async_copy(ys_hbm.at[i1_v.at[c]], buf_o, sem).wait()

        def body(r, carry):
            idx = jnp.full((16,), c * 64 + r, jnp.int32)
            ge = plsc.load_gather(g0_v, [idx])
            go = plsc.load_gather(g1_v, [idx])
            for k in range(DD // 16):
                a = buf_e[r, pl.ds(k * 16, 16)]
                b = buf_o[r, pl.ds(k * 16, 16)]
                buf_e[r, pl.ds(k * 16, 16)] = a * ge + b * go
            return carry

        lax.fori_loop(0, 64, body, jnp.int32(0))
        pltpu.sync_copy(buf_e, out_hbm.at[pl.ds(t0 + c * 64, 64)])


def _combine(ys, inv0, inv1, g0, g1):
    mesh = plsc.VectorSubcoreMesh(core_axis_name="c", subcore_axis_name="s")
    fn = pl.kernel(
        _combine_body,
        mesh=mesh,
        out_type=jax.ShapeDtypeStruct((NTOK, DD), jnp.float32),
        scratch_types=[
            pltpu.VMEM((2, 64), jnp.int32),
            pltpu.VMEM((2, 64), jnp.int32),
            pltpu.VMEM((CHUNK,), jnp.float32),
            pltpu.VMEM((CHUNK,), jnp.float32),
            pltpu.VMEM((64, DD), jnp.float32),
            pltpu.VMEM((64, DD), jnp.float32),
            pltpu.SemaphoreType.DMA,
        ],
        compiler_params=pltpu.CompilerParams(needs_layout_passes=False),
    )
    return fn(ys, inv0, inv1, g0, g1)


def kernel(inputs, Wr, W1, b1, W2, b2):
    x = inputs.reshape(NTOK, DD)
    e0, e1, g0, g1, cnt = _router(x, Wr)
    xs, inv0, inv1, eot = _route(
        e0.reshape(NTOK), e1.reshape(NTOK), cnt.reshape(NW * EE), x)
    ys = _ffn(eot, xs, W1, b1, W2, b2)
    out = _combine(ys, inv0, inv1, g0.reshape(NTOK), g1.reshape(NTOK))
    return out.reshape(BB, SS, DD)


# speedup vs baseline: 1.0652x; 1.0574x over previous
import functools

import jax
import jax.numpy as jnp
from jax import lax
from jax.experimental import pallas as pl
from jax.experimental.pallas import tpu as pltpu
from jax.experimental.pallas import tpu_sc as plsc

BB, SS, DD = 2, 2048, 768
EE = 8
FF = 1024
NTOK = BB * SS
NW = 32
CHUNK = NTOK // NW
TROW = 256
NTILES = 40
XROWS = NTILES * TROW
EOTPAD = 48


def _router_body(x_ref, wr_ref, e0_ref, e1_ref, g0_ref, g1_ref, cnt_ref):
    x = x_ref[...]
    logits = jnp.dot(x, wr_ref[...], preferred_element_type=jnp.float32)
    probs = jax.nn.softmax(logits, axis=-1)
    i1 = jnp.argmax(probs, axis=-1)[:, None]
    p1 = jnp.max(probs, axis=-1, keepdims=True)
    cols = lax.broadcasted_iota(jnp.int32, probs.shape, 1)
    masked = jnp.where(cols == i1, -jnp.inf, probs)
    i2 = jnp.argmax(masked, axis=-1)[:, None]
    p2 = jnp.max(masked, axis=-1, keepdims=True)
    e2 = jnp.exp(p2 - p1)
    e0_ref[...] = i1
    e1_ref[...] = i2
    g0_ref[...] = 1.0 / (1.0 + e2)
    g1_ref[...] = e2 / (1.0 + e2)
    onehot = ((cols == i1) | (cols == i2)).astype(jnp.float32)
    chunk_of = lax.broadcasted_iota(jnp.int32, (NTOK, NW), 0) // CHUNK
    wcol = lax.broadcasted_iota(jnp.int32, (NTOK, NW), 1)
    ind = (chunk_of == wcol).astype(jnp.float32)
    cnt = lax.dot_general(ind, onehot, (((0,), (0,)), ((), ())),
                          preferred_element_type=jnp.float32)
    cnt_ref[...] = cnt.astype(jnp.int32)


def _router(x, Wr):
    return pl.pallas_call(
        _router_body,
        grid=(1,),
        in_specs=[
            pl.BlockSpec((NTOK, DD), lambda i: (0, 0)),
            pl.BlockSpec((DD, EE), lambda i: (0, 0)),
        ],
        out_specs=[
            pl.BlockSpec((NTOK, 1), lambda i: (0, 0)),
            pl.BlockSpec((NTOK, 1), lambda i: (0, 0)),
            pl.BlockSpec((NTOK, 1), lambda i: (0, 0)),
            pl.BlockSpec((NTOK, 1), lambda i: (0, 0)),
            pl.BlockSpec((NW, EE), lambda i: (0, 0)),
        ],
        out_shape=[
            jax.ShapeDtypeStruct((NTOK, 1), jnp.int32),
            jax.ShapeDtypeStruct((NTOK, 1), jnp.int32),
            jax.ShapeDtypeStruct((NTOK, 1), jnp.float32),
            jax.ShapeDtypeStruct((NTOK, 1), jnp.float32),
            jax.ShapeDtypeStruct((NW, EE), jnp.int32),
        ],
        compiler_params=pltpu.CompilerParams(
            dimension_semantics=("arbitrary",),
            fuse_transposed_lhs_in_matmul=True,
        ),
    )(x, Wr)


def _route_body(e0_hbm, e1_hbm, cnt_hbm, x_hbm,
                xs_hbm, inv0_hbm, inv1_hbm, eot_hbm,
                e0_v, e1_v, cnt_v, pos0_v, pos1_v, xbuf, eot_v, sem):
    info = plsc.get_sparse_core_info()
    nc = info.num_cores
    w = lax.axis_index("s") * nc + lax.axis_index("c")
    t0 = w * CHUNK
    pltpu.sync_copy(e0_hbm.at[pl.ds(t0, CHUNK)], e0_v)
    pltpu.sync_copy(e1_hbm.at[pl.ds(t0, CHUNK)], e1_v)
    pltpu.sync_copy(cnt_hbm, cnt_v)

    lanes = lax.iota(jnp.int32, 16)
    starts = []
    nt = []
    base = jnp.int32(0)
    for e in range(EE):
        col_a = plsc.load_gather(cnt_v, [lanes * EE + e])
        col_b = plsc.load_gather(cnt_v, [(lanes + 16) * EE + e])
        c_e = jnp.sum(col_a) + jnp.sum(col_b)
        pref = (jnp.sum(jnp.where(lanes < w, col_a, 0))
                + jnp.sum(jnp.where(lanes + 16 < w, col_b, 0)))
        starts.append(base + pref)
        nt_e = (c_e + (TROW - 1)) // TROW
        nt.append(nt_e)
        base = base + nt_e * TROW

    @pl.when(w == 0)
    def _():
        vals = [jnp.full((16,), EE - 1, jnp.int32) for _ in range(EOTPAD // 16)]
        toff = jnp.int32(0)
        for e in range(EE):
            for k in range(EOTPAD // 16):
                tv = lanes + 16 * k
                m = (tv >= toff) & (tv < toff + nt[e])
                vals[k] = jnp.where(m, e, vals[k])
            toff = toff + nt[e]
        kx, lx = divmod(NTILES, 16)
        vals[kx] = jnp.where(lanes == lx, toff, vals[kx])
        for k in range(EOTPAD // 16):
            eot_v[pl.ds(16 * k, 16)] = vals[k]
        pltpu.sync_copy(eot_v, eot_hbm)

    for ev_ref, pos_ref in ((e0_v, pos0_v), (e1_v, pos1_v)):
        for v in range(CHUNK // 16):
            ev = ev_ref[pl.ds(v * 16, 16)]
            pos = jnp.zeros((16,), jnp.int32)
            for e in range(EE):
                m = ev == e
                pc = plsc.cumsum(jnp.where(m, 1, 0).astype(jnp.int32))
                pos = jnp.where(m, starts[e] + pc - 1, pos)
                starts[e] = starts[e] + jnp.max(pc)
            pos_ref[v // 4, pl.ds((v % 4) * 16, 16)] = pos

    pltpu.sync_copy(pos0_v, inv0_hbm.at[w])
    pltpu.sync_copy(pos1_v, inv1_hbm.at[w])

    for c in range(2):
        pltpu.sync_copy(x_hbm.at[pl.ds(t0 + c * 64, 64)], xbuf)
        pltpu.async_copy(xbuf, xs_hbm.at[pos0_v.at[c]], sem).wait()
        pltpu.async_copy(xbuf, xs_hbm.at[pos1_v.at[c]], sem).wait()


def _route(e0, e1, cnt, x):
    mesh = plsc.VectorSubcoreMesh(core_axis_name="c", subcore_axis_name="s")
    fn = pl.kernel(
        _route_body,
        mesh=mesh,
        out_type=[
            jax.ShapeDtypeStruct((XROWS, DD), jnp.float32),
            jax.ShapeDtypeStruct((NW, 2, 64), jnp.int32),
            jax.ShapeDtypeStruct((NW, 2, 64), jnp.int32),
            jax.ShapeDtypeStruct((EOTPAD,), jnp.int32),
        ],
        scratch_types=[
            pltpu.VMEM((CHUNK,), jnp.int32),
            pltpu.VMEM((CHUNK,), jnp.int32),
            pltpu.VMEM((NW * EE,), jnp.int32),
            pltpu.VMEM((2, 64), jnp.int32),
            pltpu.VMEM((2, 64), jnp.int32),
            pltpu.VMEM((64, DD), jnp.float32),
            pltpu.VMEM((EOTPAD,), jnp.int32),
            pltpu.SemaphoreType.DMA,
        ],
        compiler_params=pltpu.CompilerParams(needs_layout_passes=False),
    )
    return fn(e0, e1, cnt, x)


CSTEPS = 10
TPS = NTILES // CSTEPS
CROWS = TPS * TROW


def _ffn_body(eot_ref, xs_ref, w1_ref, b1_ref, w2_ref, b2_ref, ys_ref):
    i = pl.program_id(0)
    for t in range(TPS):
        e = eot_ref[i * TPS + t]
        x = xs_ref[pl.ds(t * TROW, TROW), :]
        h = jnp.dot(x, w1_ref[e], preferred_element_type=jnp.float32) + b1_ref[e]
        h = h * 0.5 * (1.0 + lax.erf(h * 0.7071067811865476))
        ys_ref[pl.ds(t * TROW, TROW), :] = (
            jnp.dot(h, w2_ref[e], preferred_element_type=jnp.float32) + b2_ref[e])


def _ffn(eot, xs, W1, b1, W2, b2):
    grid_spec = pltpu.PrefetchScalarGridSpec(
        num_scalar_prefetch=1,
        grid=(CSTEPS,),
        in_specs=[
            pl.BlockSpec((CROWS, DD), lambda i, eot: (i, 0)),
            pl.BlockSpec((EE, DD, FF), lambda i, eot: (0, 0, 0)),
            pl.BlockSpec((EE, FF), lambda i, eot: (0, 0)),
            pl.BlockSpec((EE, FF, DD), lambda i, eot: (0, 0, 0)),
            pl.BlockSpec((EE, DD), lambda i, eot: (0, 0)),
        ],
        out_specs=pl.BlockSpec((CROWS, DD), lambda i, eot: (i, 0)),
    )
    return pl.pallas_call(
        _ffn_body,
        grid_spec=grid_spec,
        out_shape=jax.ShapeDtypeStruct((XROWS, DD), jnp.float32),
        compiler_params=pltpu.CompilerParams(
            dimension_semantics=("arbitrary",),
            vmem_limit_bytes=134217728,
        ),
    )(eot, xs, W1, b1, W2, b2)


def _combine_body(ys_hbm, inv0_hbm, inv1_hbm, g0_hbm, g1_hbm, out_hbm,
                  i0_v, i1_v, g0_v, g1_v, buf_e, buf_o, sem):
    info = plsc.get_sparse_core_info()
    nc = info.num_cores
    w = lax.axis_index("s") * nc + lax.axis_index("c")
    t0 = w * CHUNK
    pltpu.sync_copy(inv0_hbm.at[w], i0_v)
    pltpu.sync_copy(inv1_hbm.at[w], i1_v)
    pltpu.sync_copy(g0_hbm.at[pl.ds(t0, CHUNK)], g0_v)
    pltpu.sync_copy(g1_hbm.at[pl.ds(t0, CHUNK)], g1_v)
    for c in range(2):
        pltpu.async_copy(ys_hbm.at[i0_v.at[c]], buf_e, sem).wait()
        pltpu.async_copy(ys_hbm.at[i1_v.at[c]], buf_o, sem).wait()

        def body(r, carry):
            idx = jnp.full((16,), c * 64 + r, jnp.int32)
            ge = plsc.load_gather(g0_v, [idx])
            go = plsc.load_gather(g1_v, [idx])
            for k in range(DD // 16):
                a = buf_e[r, pl.ds(k * 16, 16)]
                b = buf_o[r, pl.ds(k * 16, 16)]
                buf_e[r, pl.ds(k * 16, 16)] = a * ge + b * go
            return carry

        lax.fori_loop(0, 64, body, jnp.int32(0))
        pltpu.sync_copy(buf_e, out_hbm.at[pl.ds(t0 + c * 64, 64)])


def _combine(ys, inv0, inv1, g0, g1):
    mesh = plsc.VectorSubcoreMesh(core_axis_name="c", subcore_axis_name="s")
    fn = pl.kernel(
        _combine_body,
        mesh=mesh,
        out_type=jax.ShapeDtypeStruct((NTOK, DD), jnp.float32),
        scratch_types=[
            pltpu.VMEM((2, 64), jnp.int32),
            pltpu.VMEM((2, 64), jnp.int32),
            pltpu.VMEM((CHUNK,), jnp.float32),
            pltpu.VMEM((CHUNK,), jnp.float32),
            pltpu.VMEM((64, DD), jnp.float32),
            pltpu.VMEM((64, DD), jnp.float32),
            pltpu.SemaphoreType.DMA,
        ],
        compiler_params=pltpu.CompilerParams(needs_layout_passes=False),
    )
    return fn(ys, inv0, inv1, g0, g1)


def kernel(inputs, Wr, W1, b1, W2, b2):
    x = inputs.reshape(NTOK, DD)
    e0, e1, g0, g1, cnt = _router(x, Wr)
    xs, inv0, inv1, eot = _route(
        e0.reshape(NTOK), e1.reshape(NTOK), cnt.reshape(NW * EE), x)
    ys = _ffn(eot, xs, W1, b1, W2, b2)
    out = _combine(ys, inv0, inv1, g0.reshape(NTOK), g1.reshape(NTOK))
    return out.reshape(BB, SS, DD)


# speedup vs baseline: 1.0990x; 1.0317x over previous
import functools

import jax
import jax.numpy as jnp
from jax import lax
from jax.experimental import pallas as pl
from jax.experimental.pallas import tpu as pltpu
from jax.experimental.pallas import tpu_sc as plsc

BB, SS, DD = 2, 2048, 768
EE = 8
FF = 1024
NTOK = BB * SS
NW = 32
CHUNK = NTOK // NW
TROW = 256
NTILES = 40
XROWS = NTILES * TROW
EOTPAD = 48


def _router_body(x_ref, wr_ref, ep_ref, g0_ref, g1_ref, cnt_ref):
    x = x_ref[...]
    logits = jnp.dot(x, wr_ref[...], preferred_element_type=jnp.float32)
    probs = jax.nn.softmax(logits, axis=-1)
    i1 = jnp.argmax(probs, axis=-1)[:, None]
    p1 = jnp.max(probs, axis=-1, keepdims=True)
    cols = lax.broadcasted_iota(jnp.int32, probs.shape, 1)
    masked = jnp.where(cols == i1, -jnp.inf, probs)
    i2 = jnp.argmax(masked, axis=-1)[:, None]
    p2 = jnp.max(masked, axis=-1, keepdims=True)
    e2 = jnp.exp(p2 - p1)
    ep_ref[...] = i1 * EE + i2
    g0_ref[...] = 1.0 / (1.0 + e2)
    g1_ref[...] = e2 / (1.0 + e2)
    onehot = ((cols == i1) | (cols == i2)).astype(jnp.float32)
    chunk_of = lax.broadcasted_iota(jnp.int32, (NTOK, NW), 0) // CHUNK
    wcol = lax.broadcasted_iota(jnp.int32, (NTOK, NW), 1)
    ind = (chunk_of == wcol).astype(jnp.float32)
    cnt = lax.dot_general(ind, onehot, (((0,), (0,)), ((), ())),
                          preferred_element_type=jnp.float32)
    cnt_ref[...] = cnt.astype(jnp.int32)


def _router(x, Wr):
    return pl.pallas_call(
        _router_body,
        grid=(1,),
        in_specs=[
            pl.BlockSpec((NTOK, DD), lambda i: (0, 0)),
            pl.BlockSpec((DD, EE), lambda i: (0, 0)),
        ],
        out_specs=[
            pl.BlockSpec((NTOK, 1), lambda i: (0, 0)),
            pl.BlockSpec((NTOK, 1), lambda i: (0, 0)),
            pl.BlockSpec((NTOK, 1), lambda i: (0, 0)),
            pl.BlockSpec((NW, EE), lambda i: (0, 0)),
        ],
        out_shape=[
            jax.ShapeDtypeStruct((NTOK, 1), jnp.int32),
            jax.ShapeDtypeStruct((NTOK, 1), jnp.float32),
            jax.ShapeDtypeStruct((NTOK, 1), jnp.float32),
            jax.ShapeDtypeStruct((NW, EE), jnp.int32),
        ],
        compiler_params=pltpu.CompilerParams(
            dimension_semantics=("arbitrary",),
            fuse_transposed_lhs_in_matmul=True,
        ),
    )(x, Wr)


def _route_body(ep_hbm, cnt_hbm, x_hbm,
                xs_hbm, inv0_hbm, inv1_hbm, eot_hbm,
                ep_v, cnt_v, pos0_v, pos1_v, xbuf, eot_v, sem):
    info = plsc.get_sparse_core_info()
    nc = info.num_cores
    w = lax.axis_index("s") * nc + lax.axis_index("c")
    t0 = w * CHUNK
    pltpu.sync_copy(ep_hbm.at[pl.ds(t0, CHUNK)], ep_v)
    pltpu.sync_copy(cnt_hbm, cnt_v)

    lanes = lax.iota(jnp.int32, 16)
    starts = []
    nt = []
    base = jnp.int32(0)
    for e in range(EE):
        col_a = plsc.load_gather(cnt_v, [lanes * EE + e])
        col_b = plsc.load_gather(cnt_v, [(lanes + 16) * EE + e])
        c_e = jnp.sum(col_a) + jnp.sum(col_b)
        pref = (jnp.sum(jnp.where(lanes < w, col_a, 0))
                + jnp.sum(jnp.where(lanes + 16 < w, col_b, 0)))
        starts.append(base + pref)
        nt_e = (c_e + (TROW - 1)) // TROW
        nt.append(nt_e)
        base = base + nt_e * TROW

    @pl.when(w == 0)
    def _():
        vals = [jnp.full((16,), EE - 1, jnp.int32) for _ in range(EOTPAD // 16)]
        toff = jnp.int32(0)
        for e in range(EE):
            for k in range(EOTPAD // 16):
                tv = lanes + 16 * k
                m = (tv >= toff) & (tv < toff + nt[e])
                vals[k] = jnp.where(m, e, vals[k])
            toff = toff + nt[e]
        kx, lx = divmod(NTILES, 16)
        vals[kx] = jnp.where(lanes == lx, toff, vals[kx])
        for k in range(EOTPAD // 16):
            eot_v[pl.ds(16 * k, 16)] = vals[k]
        pltpu.sync_copy(eot_v, eot_hbm)

    for slot, pos_ref in ((0, pos0_v), (1, pos1_v)):
        for v in range(CHUNK // 16):
            epv = ep_v[pl.ds(v * 16, 16)]
            ev = jnp.where(slot == 0, epv // EE, epv % EE)
            pos = jnp.zeros((16,), jnp.int32)
            for e in range(EE):
                m = ev == e
                pc = plsc.cumsum(jnp.where(m, 1, 0).astype(jnp.int32))
                pos = jnp.where(m, starts[e] + pc - 1, pos)
                starts[e] = starts[e] + jnp.max(pc)
            pos_ref[v // 4, pl.ds((v % 4) * 16, 16)] = pos

    pltpu.sync_copy(pos0_v, inv0_hbm.at[w])
    pltpu.sync_copy(pos1_v, inv1_hbm.at[w])

    for c in range(2):
        pltpu.sync_copy(x_hbm.at[pl.ds(t0 + c * 64, 64)], xbuf)
        pltpu.async_copy(xbuf, xs_hbm.at[pos0_v.at[c]], sem).wait()
        pltpu.async_copy(xbuf, xs_hbm.at[pos1_v.at[c]], sem).wait()


def _route(ep, cnt, x):
    mesh = plsc.VectorSubcoreMesh(core_axis_name="c", subcore_axis_name="s")
    fn = pl.kernel(
        _route_body,
        mesh=mesh,
        out_type=[
            jax.ShapeDtypeStruct((XROWS, DD), jnp.float32),
            jax.ShapeDtypeStruct((NW, 2, 64), jnp.int32),
            jax.ShapeDtypeStruct((NW, 2, 64), jnp.int32),
            jax.ShapeDtypeStruct((EOTPAD,), jnp.int32),
        ],
        scratch_types=[
            pltpu.VMEM((CHUNK,), jnp.int32),
            pltpu.VMEM((NW * EE,), jnp.int32),
            pltpu.VMEM((2, 64), jnp.int32),
            pltpu.VMEM((2, 64), jnp.int32),
            pltpu.VMEM((64, DD), jnp.float32),
            pltpu.VMEM((EOTPAD,), jnp.int32),
            pltpu.SemaphoreType.DMA,
        ],
        compiler_params=pltpu.CompilerParams(needs_layout_passes=False),
    )
    return fn(ep, cnt, x)


CSTEPS = 10
TPS = NTILES // CSTEPS
CROWS = TPS * TROW


def _ffn_body(eot_ref, xs_ref, w1_ref, b1_ref, w2_ref, b2_ref, ys_ref):
    i = pl.program_id(0)
    for t in range(TPS):
        e = eot_ref[i * TPS + t]
        x = xs_ref[pl.ds(t * TROW, TROW), :]
        h = jnp.dot(x, w1_ref[e], preferred_element_type=jnp.float32) + b1_ref[e]
        h = h * 0.5 * (1.0 + lax.erf(h * 0.7071067811865476))
        ys_ref[pl.ds(t * TROW, TROW), :] = (
            jnp.dot(h, w2_ref[e], preferred_element_type=jnp.float32) + b2_ref[e])


def _ffn(eot, xs, W1, b1, W2, b2):
    grid_spec = pltpu.PrefetchScalarGridSpec(
        num_scalar_prefetch=1,
        grid=(CSTEPS,),
        in_specs=[
            pl.BlockSpec((CROWS, DD), lambda i, eot: (i, 0)),
            pl.BlockSpec((EE, DD, FF), lambda i, eot: (0, 0, 0)),
            pl.BlockSpec((EE, FF), lambda i, eot: (0, 0)),
            pl.BlockSpec((EE, FF, DD), lambda i, eot: (0, 0, 0)),
            pl.BlockSpec((EE, DD), lambda i, eot: (0, 0)),
        ],
        out_specs=pl.BlockSpec((CROWS, DD), lambda i, eot: (i, 0)),
    )
    return pl.pallas_call(
        _ffn_body,
        grid_spec=grid_spec,
        out_shape=jax.ShapeDtypeStruct((XROWS, DD), jnp.float32),
        compiler_params=pltpu.CompilerParams(
            dimension_semantics=("arbitrary",),
            vmem_limit_bytes=134217728,
        ),
    )(eot, xs, W1, b1, W2, b2)


def _combine_body(ys_hbm, inv0_hbm, inv1_hbm, g0_hbm, g1_hbm, out_hbm,
                  i0_v, i1_v, g0_v, g1_v, buf_e, buf_o, sem, sem2):
    info = plsc.get_sparse_core_info()
    nc = info.num_cores
    w = lax.axis_index("s") * nc + lax.axis_index("c")
    t0 = w * CHUNK
    pltpu.sync_copy(inv0_hbm.at[w], i0_v)
    pltpu.sync_copy(inv1_hbm.at[w], i1_v)
    pltpu.sync_copy(g0_hbm.at[pl.ds(t0, CHUNK)], g0_v)
    pltpu.sync_copy(g1_hbm.at[pl.ds(t0, CHUNK)], g1_v)
    for c in range(2):
        cp0 = pltpu.async_copy(ys_hbm.at[i0_v.at[c]], buf_e, sem)
        cp1 = pltpu.async_copy(ys_hbm.at[i1_v.at[c]], buf_o, sem2)
        cp0.wait()
        cp1.wait()

        def body(r, carry):
            idx = jnp.full((16,), c * 64 + r, jnp.int32)
            ge = plsc.load_gather(g0_v, [idx])
            go = plsc.load_gather(g1_v, [idx])
            for k in range(DD // 16):
                a = buf_e[r, pl.ds(k * 16, 16)]
                b = buf_o[r, pl.ds(k * 16, 16)]
                buf_e[r, pl.ds(k * 16, 16)] = a * ge + b * go
            return carry

        lax.fori_loop(0, 64, body, jnp.int32(0))
        pltpu.sync_copy(buf_e, out_hbm.at[pl.ds(t0 + c * 64, 64)])


def _combine(ys, inv0, inv1, g0, g1):
    mesh = plsc.VectorSubcoreMesh(core_axis_name="c", subcore_axis_name="s")
    fn = pl.kernel(
        _combine_body,
        mesh=mesh,
        out_type=jax.ShapeDtypeStruct((NTOK, DD), jnp.float32),
        scratch_types=[
            pltpu.VMEM((2, 64), jnp.int32),
            pltpu.VMEM((2, 64), jnp.int32),
            pltpu.VMEM((CHUNK,), jnp.float32),
            pltpu.VMEM((CHUNK,), jnp.float32),
            pltpu.VMEM((64, DD), jnp.float32),
            pltpu.VMEM((64, DD), jnp.float32),
            pltpu.SemaphoreType.DMA,
            pltpu.SemaphoreType.DMA,
        ],
        compiler_params=pltpu.CompilerParams(needs_layout_passes=False),
    )
    return fn(ys, inv0, inv1, g0, g1)


def kernel(inputs, Wr, W1, b1, W2, b2):
    x = inputs.reshape(NTOK, DD)
    ep, g0, g1, cnt = _router(x, Wr)
    xs, inv0, inv1, eot = _route(ep.reshape(NTOK), cnt.reshape(NW * EE), x)
    ys = _ffn(eot, xs, W1, b1, W2, b2)
    out = _combine(ys, inv0, inv1, g0.reshape(NTOK), g1.reshape(NTOK))
    return out.reshape(BB, SS, DD)
